# baseline (speedup 1.0000x reference)
import jax
import jax.numpy as jnp
from jax import lax
from jax.experimental import pallas as pl
from jax.experimental.pallas import tpu as pltpu
from jax.experimental.pallas import tpu_sc as plsc

N_USER = 5000
N_ITEM = 5000
N = N_USER + N_ITEM
D = 128
NC = 2
NS = 16
W = NC * NS
L = 16
NPAD = 10240
ROWS_PER_TILE = NPAD // NS
K = 128
NROWCHUNKS = ROWS_PER_TILE // K
E = 320000
EPW = E // W
NCH = 80
NGCH = W * NCH + 2

_SC_PARAMS = pltpu.CompilerParams(
    use_tc_tiling_on_sc=False, needs_layout_passes=False)


def _zero_acc_slice(gb1, acc, rbase):
    def zrow(r, _):
        for j in range(D // L):
            gb1[r, pl.ds(j * L, L)] = jnp.zeros((L,), jnp.float32)
        return 0
    lax.fori_loop(0, K, zrow, 0)

    def zchunk(i, _):
        pltpu.sync_copy(gb1.at[pl.ds(0, K)],
                        acc.at[pl.ds(rbase + i * K, K)])
        return 0
    lax.fori_loop(0, NROWCHUNKS, zchunk, 0)


def _spmm_layer(src_hbm, edg, acc, c, gbase, ib, gb, gsem, ssem, isem):
    def start_gather(t_ph, tb):
        pltpu.async_copy(src_hbm.at[ib[t_ph].at[0]], gb[tb], gsem[tb])

    def start_scatter(t_ph, tb):
        pltpu.async_copy(gb[tb], acc.at[ib[t_ph].at[1]], ssem[tb], add=True)

    def wait_gather(tb):
        pltpu.make_async_copy(src_hbm.at[pl.ds(0, K)], gb[tb],
                              gsem[tb]).wait()

    def wait_scatter(tb):
        pltpu.make_async_copy(gb[tb], acc.at[pl.ds(0, K)], ssem[tb]).wait()

    def wait_idx(i4):
        pltpu.make_async_copy(edg.at[c, gbase], ib[i4], isem[i4]).wait()

    def scale(gbuf, ibuf):
        @plsc.parallel_loop(0, K // L, unroll=2)
        def group(g):
            val16 = plsc.bitcast(ibuf[2, pl.ds(g * L, L)], jnp.float32)
            for el in range(L):
                e = g * L + el
                v = val16[el]
                for j in range(D // L):
                    sl = pl.ds(j * L, L)
                    gbuf[e, sl] = gbuf[e, sl] * v

    pltpu.sync_copy(edg.at[c, gbase], ib[0])
    pltpu.sync_copy(edg.at[c, gbase + 1], ib[1])
    start_gather(0, 0)
    start_gather(1, 1)
    pltpu.async_copy(edg.at[c, gbase + 2], ib[2], isem[2])
    pltpu.async_copy(edg.at[c, gbase + 3], ib[3], isem[3])
    wait_gather(0)
    scale(gb[0], ib[0])
    start_scatter(0, 0)

    def step(t, tm):
        b, p = tm % 2, (tm - 1) % 2
        i4, i4p, i4n = tm % 4, (tm - 1) % 4, (tm + 2) % 4
        wait_idx(i4)
        wait_scatter(b)
        start_gather(i4, b)
        pltpu.async_copy(edg.at[c, gbase + t + 2], ib[i4n], isem[i4n])
        wait_gather(p)
        scale(gb[p], ib[i4p])
        start_scatter(i4p, p)

    def quad(q, _):
        t0 = 2 + 4 * q
        for k in range(4):
            step(t0 + k, 2 + k)
        return 0
    lax.fori_loop(0, (NCH - 4) // 4, quad, 0)
    step(NCH - 2, NCH - 2)
    step(NCH - 1, NCH - 1)

    wait_gather((NCH - 1) % 2)
    scale(gb[(NCH - 1) % 2], ib[(NCH - 1) % 4])
    start_scatter((NCH - 1) % 4, (NCH - 1) % 2)
    wait_scatter((NCH - 2) % 2)
    wait_scatter((NCH - 1) % 2)
    wait_idx(NCH % 4)
    wait_idx((NCH + 1) % 4)


def _export_acc(acc, dst, rbase, hbase):
    def exp(i, _):
        r0 = rbase + i * K
        pltpu.sync_copy(acc.at[pl.ds(r0, K)],
                        dst.at[pl.ds(hbase + r0, K)])
        return 0
    lax.fori_loop(0, NROWCHUNKS, exp, 0)


_SPMM_SCRATCH = [
    pltpu.VMEM_SHARED((NPAD, D), jnp.float32),
    pltpu.VMEM((3, K), jnp.int32),
    pltpu.VMEM((3, K), jnp.int32),
    pltpu.VMEM((3, K), jnp.int32),
    pltpu.VMEM((3, K), jnp.int32),
    pltpu.VMEM((K, D), jnp.float32),
    pltpu.VMEM((K, D), jnp.float32),
    pltpu.SemaphoreType.DMA,
    pltpu.SemaphoreType.DMA,
    pltpu.SemaphoreType.DMA,
    pltpu.SemaphoreType.DMA,
    pltpu.SemaphoreType.DMA,
    pltpu.SemaphoreType.DMA,
    pltpu.SemaphoreType.DMA,
    pltpu.SemaphoreType.DMA,
]


def _layer1_body(x0, edg, p1, acc,
                 ib0, ib1, ib2, ib3, gb0, gb1,
                 gsem0, gsem1, ssem0, ssem1, isem0, isem1, isem2, isem3):
    c = lax.axis_index("c")
    s = lax.axis_index("s")
    w = s * NC + c
    rbase = s * ROWS_PER_TILE
    hbase = c * NPAD

    _zero_acc_slice(gb1, acc, rbase)
    plsc.subcore_barrier()
    _spmm_layer(x0, edg, acc, c, w * NCH,
                (ib0, ib1, ib2, ib3), (gb0, gb1),
                (gsem0, gsem1), (ssem0, ssem1),
                (isem0, isem1, isem2, isem3))
    plsc.subcore_barrier()
    _export_acc(acc, p1, rbase, hbase)


def _layer2_body(p1, edg, p2, e1, acc,
                 ib0, ib1, ib2, ib3, gb0, gb1,
                 gsem0, gsem1, ssem0, ssem1, isem0, isem1, isem2, isem3):
    c = lax.axis_index("c")
    s = lax.axis_index("s")
    w = s * NC + c
    rbase = s * ROWS_PER_TILE
    hbase = c * NPAD

    def build(i, _):
        r0 = rbase + i * K
        pltpu.sync_copy(p1.at[pl.ds(r0, K)], gb0.at[pl.ds(0, K)])
        pltpu.sync_copy(p1.at[pl.ds(NPAD + r0, K)], gb1.at[pl.ds(0, K)])

        @plsc.parallel_loop(0, K, unroll=2)
        def addrow(r):
            for j in range(D // L):
                sl = pl.ds(j * L, L)
                gb0[r, sl] = gb0[r, sl] + gb1[r, sl]
        pltpu.sync_copy(gb0.at[pl.ds(0, K)],
                        e1.at[pl.ds(hbase + r0, K)])
        return 0
    lax.fori_loop(0, NROWCHUNKS, build, 0)
    _zero_acc_slice(gb1, acc, rbase)
    plsc.subcore_barrier()
    _spmm_layer(e1, edg, acc, c, w * NCH,
                (ib0, ib1, ib2, ib3), (gb0, gb1),
                (gsem0, gsem1), (ssem0, ssem1),
                (isem0, isem1, isem2, isem3))
    plsc.subcore_barrier()
    _export_acc(acc, p2, rbase, hbase)


def _final_body(x0, e1, p2, out, b0, b1, b2, b3):
    c = lax.axis_index("c")
    s = lax.axis_index("s")
    w = s * NC + c
    FR = NPAD // W
    CH = 64

    def chunk(i, _):
        r0 = w * FR + i * CH
        pltpu.sync_copy(x0.at[pl.ds(r0, CH)], b0)
        pltpu.sync_copy(e1.at[pl.ds(r0, CH)], b1)
        pltpu.sync_copy(p2.at[pl.ds(r0, CH)], b2)
        pltpu.sync_copy(p2.at[pl.ds(NPAD + r0, CH)], b3)

        @plsc.parallel_loop(0, CH, unroll=2)
        def addrow(r):
            for j in range(D // L):
                sl = pl.ds(j * L, L)
                b0[r, sl] = ((b0[r, sl] + b1[r, sl])
                             + (b2[r, sl] + b3[r, sl]))
        pltpu.sync_copy(b0, out.at[pl.ds(r0, CH)])
        return 0
    lax.fori_loop(0, FR // CH, chunk, 0)


@jax.jit
def _run(x0, edg1, edg2):
    mesh = plsc.VectorSubcoreMesh(
        core_axis_name="c", subcore_axis_name="s",
        num_cores=NC, num_subcores=NS)
    p1 = pl.kernel(
        _layer1_body,
        out_type=jax.ShapeDtypeStruct((NC * NPAD, D), jnp.float32),
        mesh=mesh, compiler_params=_SC_PARAMS,
        scratch_types=list(_SPMM_SCRATCH),
    )(x0, edg1)
    p2, e1 = pl.kernel(
        _layer2_body,
        out_type=[
            jax.ShapeDtypeStruct((NC * NPAD, D), jnp.float32),
            jax.ShapeDtypeStruct((NC * NPAD, D), jnp.float32),
        ],
        mesh=mesh, compiler_params=_SC_PARAMS,
        scratch_types=list(_SPMM_SCRATCH),
    )(p1, edg2)
    out = pl.kernel(
        _final_body,
        out_type=jax.ShapeDtypeStruct((NPAD, D), jnp.float32),
        mesh=mesh, compiler_params=_SC_PARAMS,
        scratch_types=[
            pltpu.VMEM((64, D), jnp.float32),
            pltpu.VMEM((64, D), jnp.float32),
            pltpu.VMEM((64, D), jnp.float32),
            pltpu.VMEM((64, D), jnp.float32),
        ],
    )(x0, e1, p2)
    return out


def kernel(adj_indices, adj_values, uEmbeds, iEmbeds):
    emb = jnp.concatenate([uEmbeds, iEmbeds], axis=0)
    x0 = jnp.zeros((NPAD, D), jnp.float32).at[:N].set(emb)

    rows = adj_indices[0].astype(jnp.int32)
    cols = adj_indices[1].astype(jnp.int32)
    vals_i = lax.bitcast_convert_type(adj_values.astype(jnp.float32),
                                      jnp.int32)
    pad = ((0, 0), (0, NCH * K - EPW))
    shp = (W, NCH, K)
    rows_p = jnp.pad(rows.reshape(W, EPW), pad).reshape(shp)
    cols_p = jnp.pad(cols.reshape(W, EPW), pad).reshape(shp)
    vals_p = jnp.pad(vals_i.reshape(W, EPW), pad).reshape(shp)

    def pack(col_chunks):
        e = jnp.stack([col_chunks, rows_p, vals_p], axis=2)
        e = e.reshape(W * NCH, 3, K)
        return jnp.pad(e, ((0, 2), (0, 0), (0, 0)))

    base = pack(cols_p)
    edg1 = jnp.stack([base, base])
    edg2 = jnp.stack([base, pack(cols_p + NPAD)])

    out = _run(x0, edg1, edg2)
    total = out[:N]
    return total[:N_USER], total[N_USER:]


# speedup vs baseline: 2.1442x; 2.1442x over previous
import jax
import jax.numpy as jnp
from jax import lax
from jax.experimental import pallas as pl
from jax.experimental.pallas import tpu as pltpu
from jax.experimental.pallas import tpu_sc as plsc

N_USER = 5000
N_ITEM = 5000
N = N_USER + N_ITEM
D = 128
DH = D // 2
NC = 2
NS = 16
L = 16
NPAD = 10240
ROWS_PER_TILE = NPAD // NS
K = 128
NROWCHUNKS = ROWS_PER_TILE // K
KO = 256
NSUB = KO // K
E = 320000
EPT = E // NS
NCH = 80
NGCH = NS * NCH + 2


def _body(x0, edg, out, xsp, esp,
          ib0, ib1, ib2, ib3, gb0, gb1,
          gsem0, gsem1, ssem0, ssem1, isem0, isem1, isem2, isem3):
    ib = (ib0, ib1, ib2, ib3)
    gb = (gb0, gb1)
    gsem = (gsem0, gsem1)
    ssem = (ssem0, ssem1)
    isem = (isem0, isem1, isem2, isem3)

    c = lax.axis_index("c")
    s = lax.axis_index("s")
    rbase = s * ROWS_PER_TILE
    gbase = s * NCH
    hbase = c * NPAD

    pltpu.sync_copy(x0.at[pl.ds(hbase + rbase, ROWS_PER_TILE)],
                    xsp.at[pl.ds(rbase, ROWS_PER_TILE)])

    def zrow(r, _):
        for j in range(DH // L):
            gb1[r, pl.ds(j * L, L)] = jnp.zeros((L,), jnp.float32)
        return 0
    lax.fori_loop(0, K, zrow, 0)

    def zchunk(i, _):
        pltpu.sync_copy(gb1.at[pl.ds(0, K)],
                        esp.at[pl.ds(rbase + i * K, K)])
        return 0
    lax.fori_loop(0, NROWCHUNKS, zchunk, 0)
    plsc.subcore_barrier()

    def scale(gbuf, ibuf):
        @plsc.parallel_loop(0, KO // L, unroll=2)
        def group(g):
            val16 = plsc.bitcast(
                ibuf[2 * NSUB + g // (K // L), pl.ds((g % (K // L)) * L, L)],
                jnp.float32)
            for el in range(L):
                e = g * L + el
                v = val16[el]
                for j in range(DH // L):
                    sl = pl.ds(j * L, L)
                    gbuf[e, sl] = gbuf[e, sl] * v

    def layer(src_sp, acc):
        def start_gathers(t_ph, tb):
            for j in range(NSUB):
                pltpu.async_copy(src_sp.at[ib[t_ph].at[j]],
                                 gb[tb].at[pl.ds(j * K, K)], gsem[tb])

        def start_scatters(t_ph, tb):
            for j in range(NSUB):
                pltpu.async_copy(gb[tb].at[pl.ds(j * K, K)],
                                 acc.at[ib[t_ph].at[NSUB + j]],
                                 ssem[tb], add=True)

        def wait_gathers(tb):
            pltpu.make_async_copy(src_sp.at[pl.ds(0, KO)], gb[tb],
                                  gsem[tb]).wait()

        def wait_scatters(tb):
            pltpu.make_async_copy(gb[tb], acc.at[pl.ds(0, KO)],
                                  ssem[tb]).wait()

        def wait_idx(i4):
            pltpu.make_async_copy(edg.at[gbase], ib[i4], isem[i4]).wait()

        pltpu.sync_copy(edg.at[gbase], ib[0])
        pltpu.sync_copy(edg.at[gbase + 1], ib[1])
        start_gathers(0, 0)
        start_gathers(1, 1)
        pltpu.async_copy(edg.at[gbase + 2], ib[2], isem[2])
        pltpu.async_copy(edg.at[gbase + 3], ib[3], isem[3])
        wait_gathers(0)
        scale(gb[0], ib[0])
        start_scatters(0, 0)

        def step(t, tm):
            b, p = tm % 2, (tm - 1) % 2
            i4, i4p, i4n = tm % 4, (tm - 1) % 4, (tm + 2) % 4
            wait_idx(i4)
            wait_scatters(b)
            start_gathers(i4, b)
            pltpu.async_copy(edg.at[gbase + t + 2], ib[i4n], isem[i4n])
            wait_gathers(p)
            scale(gb[p], ib[i4p])
            start_scatters(i4p, p)

        def quad(q, _):
            t0 = 2 + 4 * q
            for k in range(4):
                step(t0 + k, 2 + k)
            return 0
        lax.fori_loop(0, (NCH - 4) // 4, quad, 0)
        step(NCH - 2, NCH - 2)
        step(NCH - 1, NCH - 1)

        wait_gathers((NCH - 1) % 2)
        scale(gb[(NCH - 1) % 2], ib[(NCH - 1) % 4])
        start_scatters((NCH - 1) % 4, (NCH - 1) % 2)
        wait_scatters((NCH - 2) % 2)
        wait_scatters((NCH - 1) % 2)
        wait_idx(NCH % 4)
        wait_idx((NCH + 1) % 4)

    layer(xsp, esp)
    plsc.subcore_barrier()

    def mid(i, _):
        r0 = rbase + i * K
        pltpu.sync_copy(xsp.at[pl.ds(r0, K)], gb0.at[pl.ds(0, K)])
        pltpu.sync_copy(esp.at[pl.ds(r0, K)], gb1.at[pl.ds(0, K)])

        @plsc.parallel_loop(0, K, unroll=2)
        def addrow(r):
            for j in range(DH // L):
                sl = pl.ds(j * L, L)
                gb0[r, sl] = gb0[r, sl] + gb1[r, sl]
        pltpu.sync_copy(gb0.at[pl.ds(0, K)], xsp.at[pl.ds(r0, K)])
        return 0
    lax.fori_loop(0, NROWCHUNKS, mid, 0)
    plsc.subcore_barrier()

    layer(esp, xsp)
    plsc.subcore_barrier()

    pltpu.sync_copy(xsp.at[pl.ds(rbase, ROWS_PER_TILE)],
                    out.at[pl.ds(hbase + rbase, ROWS_PER_TILE)])


@jax.jit
def _run(x0, edg):
    mesh = plsc.VectorSubcoreMesh(
        core_axis_name="c", subcore_axis_name="s",
        num_cores=NC, num_subcores=NS)
    kfn = pl.kernel(
        _body,
        out_type=jax.ShapeDtypeStruct((NC * NPAD, DH), jnp.float32),
        mesh=mesh,
        compiler_params=pltpu.CompilerParams(
            use_tc_tiling_on_sc=False, needs_layout_passes=False),
        scratch_types=[
            pltpu.VMEM_SHARED((NPAD, DH), jnp.float32),
            pltpu.VMEM_SHARED((NPAD, DH), jnp.float32),
            pltpu.VMEM((3 * NSUB, K), jnp.int32),
            pltpu.VMEM((3 * NSUB, K), jnp.int32),
            pltpu.VMEM((3 * NSUB, K), jnp.int32),
            pltpu.VMEM((3 * NSUB, K), jnp.int32),
            pltpu.VMEM((KO, DH), jnp.float32),
            pltpu.VMEM((KO, DH), jnp.float32),
            pltpu.SemaphoreType.DMA,
            pltpu.SemaphoreType.DMA,
            pltpu.SemaphoreType.DMA,
            pltpu.SemaphoreType.DMA,
            pltpu.SemaphoreType.DMA,
            pltpu.SemaphoreType.DMA,
            pltpu.SemaphoreType.DMA,
            pltpu.SemaphoreType.DMA,
        ],
    )
    return kfn(x0, edg)


def kernel(adj_indices, adj_values, uEmbeds, iEmbeds):
    emb = jnp.concatenate([uEmbeds, iEmbeds], axis=0)
    embp = jnp.zeros((NPAD, D), jnp.float32).at[:N].set(emb)
    x0 = jnp.concatenate([embp[:, :DH], embp[:, DH:]], axis=0)

    rows = adj_indices[0].astype(jnp.int32)
    cols = adj_indices[1].astype(jnp.int32)
    vals_i = lax.bitcast_convert_type(adj_values.astype(jnp.float32),
                                      jnp.int32)
    pad = ((0, 0), (0, NCH * KO - EPT))
    shp = (NS, NCH, NSUB, K)
    rows_p = jnp.pad(rows.reshape(NS, EPT), pad).reshape(shp)
    cols_p = jnp.pad(cols.reshape(NS, EPT), pad).reshape(shp)
    vals_p = jnp.pad(vals_i.reshape(NS, EPT), pad).reshape(shp)
    e = jnp.concatenate([cols_p, rows_p, vals_p], axis=2)
    e = e.reshape(NS * NCH, 3 * NSUB, K)
    edg = jnp.pad(e, ((0, 2), (0, 0), (0, 0)))

    out = _run(x0, edg)
    total = jnp.concatenate([out[:N], out[NPAD:NPAD + N]], axis=1)
    return total[:N_USER], total[N_USER:]
